# baseline (speedup 1.0000x reference)
import jax
import jax.numpy as jnp
from jax import lax
from jax.experimental import pallas as pl
from jax.experimental.pallas import tpu as pltpu
from jax.experimental.pallas import tpu_sc as plsc

N = 320000
D = 128
S = 10000
NC = 2
NS = 16
W = NC * NS
SPW = 320
SPAD = W * SPW
C = 80
B = 16
NBUF = 4
NSL = D // 16
OD = 2 * D


def _tree(vals, op):
    while len(vals) > 1:
        vals = [op(vals[t], vals[t + 1]) for t in range(0, len(vals), 2)]
    return vals[0]


def _sc_body(feat_hbm, mem_hbm, bnd_hbm, out_hbm, rows_v, mem_v, bnd_v, acc,
             st_s, st_m, cur_sm, sem_r, sem_m):
    wid = lax.axis_index("s") * NC + lax.axis_index("c")

    pltpu.sync_copy(bnd_hbm.at[wid], bnd_v)
    bvec = bnd_v[...]
    lo = bvec[0]
    hi = bvec[1]

    zero = jnp.zeros((16,), jnp.float32)
    CD = C * D
    CM = C + 16

    def issue(k, buf):
        pltpu.async_copy(feat_hbm.at[pl.ds(k * CD, CD)],
                         rows_v.at[pl.ds(buf * CD, CD)], sem_r)
        pltpu.async_copy(mem_hbm.at[pl.ds(k * C, C)],
                         mem_v.at[pl.ds(buf * CM, C)], sem_m)

    def wait(k, buf):
        pltpu.make_async_copy(feat_hbm.at[pl.ds(k * CD, CD)],
                              rows_v.at[pl.ds(buf * CD, CD)], sem_r).wait()
        pltpu.make_async_copy(mem_hbm.at[pl.ds(k * C, C)],
                              mem_v.at[pl.ds(buf * CM, C)], sem_m).wait()

    def flush_regs(cur, s_regs, m_regs):
        r = cur - wid * SPW
        for j in range(NSL):
            acc[pl.ds(r * OD + 16 * j, 16)] = jnp.maximum(s_regs[j], 0.0)
            acc[pl.ds(r * OD + D + 16 * j, 16)] = jnp.maximum(m_regs[j], 0.0)

    def flush_state(cur):
        flush_regs(cur,
                   [st_s[pl.ds(16 * j, 16)] for j in range(NSL)],
                   [st_m[pl.ds(16 * j, 16)] for j in range(NSL)])

    def make_row_step(rbase, mbase):
        def row_step(i, _):
            m = mem_v[pl.ds(mbase + i, 16)][0]
            cur = cur_sm[0]
            changed = m != cur

            @pl.when(jnp.logical_and(changed, cur >= 0))
            def _():
                flush_state(cur)

            for j in range(NSL):
                row = rows_v[pl.ds(rbase + i * D + 16 * j, 16)]
                s = st_s[pl.ds(16 * j, 16)]
                mx = st_m[pl.ds(16 * j, 16)]
                st_s[pl.ds(16 * j, 16)] = jnp.where(changed, row, s + row)
                st_m[pl.ds(16 * j, 16)] = jnp.where(
                    changed, row, jnp.maximum(mx, row))
            cur_sm[0] = m
            return 0
        return row_step

    def make_block_step(rbase, mbase):
        def block_step(ib, _):
            b0 = ib * B
            mvec = mem_v[pl.ds(mbase + b0, 16)]
            cur = cur_sm[0]
            last = mvec[15]

            @pl.when(last == cur)
            def _fast():
                for j in range(NSL):
                    vals = [rows_v[pl.ds(rbase + (b0 + i) * D + 16 * j, 16)]
                            for i in range(B)]
                    st_s[pl.ds(16 * j, 16)] = (
                        st_s[pl.ds(16 * j, 16)] + _tree(vals, lambda a, b: a + b))
                    st_m[pl.ds(16 * j, 16)] = jnp.maximum(
                        st_m[pl.ds(16 * j, 16)], _tree(vals, jnp.maximum))

            @pl.when(last != cur)
            def _slow():
                s_regs = [st_s[pl.ds(16 * j, 16)] for j in range(NSL)]
                m_regs = [st_m[pl.ds(16 * j, 16)] for j in range(NSL)]
                c = cur
                for i in range(B):
                    mi = mvec[i]
                    changed = mi != c
                    rows = [rows_v[pl.ds(rbase + (b0 + i) * D + 16 * j, 16)]
                            for j in range(NSL)]

                    @pl.when(jnp.logical_and(changed, c >= 0))
                    def _(c=c, s_regs=s_regs, m_regs=m_regs):
                        flush_regs(c, s_regs, m_regs)

                    s_regs = [jnp.where(changed, rows[j], s_regs[j] + rows[j])
                              for j in range(NSL)]
                    m_regs = [jnp.where(changed, rows[j],
                                        jnp.maximum(m_regs[j], rows[j]))
                              for j in range(NSL)]
                    c = mi
                for j in range(NSL):
                    st_s[pl.ds(16 * j, 16)] = s_regs[j]
                    st_m[pl.ds(16 * j, 16)] = m_regs[j]
                cur_sm[0] = last
            return 0
        return block_step

    def chunk_body(k, _):
        buf = k % NBUF
        start = k * C
        wait(k, buf)

        @pl.when(k + NBUF - 1 < c1)
        def _():
            issue(k + NBUF - 1, (k + NBUF - 1) % NBUF)

        i0 = jnp.maximum(lo, start) - start
        i1 = jnp.minimum(hi, start + C) - start
        rbase = buf * CD
        mbase = buf * CM
        row_step = make_row_step(rbase, mbase)
        block_step = make_block_step(rbase, mbase)
        lead_end = jnp.minimum(i1, ((i0 + B - 1) // B) * B)
        lax.fori_loop(i0, lead_end, row_step, 0)
        lax.fori_loop(lead_end // B, i1 // B, block_step, 0)
        lax.fori_loop(jnp.maximum(lead_end, (i1 // B) * B), i1, row_step, 0)
        return 0

    c0 = lo // C
    c1 = (hi + C - 1) // C

    for p in range(NBUF - 1):
        @pl.when(c0 + p < c1)
        def _(p=p):
            issue(c0 + p, (c0 + p) % NBUF)

    def zero_row(r, _):
        for j in range(NSL * 2):
            acc[pl.ds(r * OD + 16 * j, 16)] = zero
        return 0

    lax.fori_loop(0, SPW, zero_row, 0)
    cur_sm[0] = jnp.int32(-1)

    lax.fori_loop(c0, c1, chunk_body, 0)

    cur = cur_sm[0]

    @pl.when(cur >= 0)
    def _():
        flush_state(cur)

    pltpu.sync_copy(acc, out_hbm.at[pl.ds(wid * (SPW * OD), SPW * OD)])


@jax.jit
def _graph_gather(feat_flat, membership, bounds2):
    mesh = plsc.VectorSubcoreMesh(
        core_axis_name="c", subcore_axis_name="s",
        num_cores=NC, num_subcores=NS)
    k = pl.kernel(
        _sc_body,
        out_type=jax.ShapeDtypeStruct((SPAD * OD,), jnp.float32),
        mesh=mesh,
        scratch_types=[
            pltpu.VMEM((NBUF * C * D,), jnp.float32),
            pltpu.VMEM((NBUF * (C + 16),), jnp.int32),
            pltpu.VMEM((16,), jnp.int32),
            pltpu.VMEM((SPW * OD,), jnp.float32),
            pltpu.VMEM((D,), jnp.float32),
            pltpu.VMEM((D,), jnp.float32),
            pltpu.SMEM((8,), jnp.int32),
            pltpu.SemaphoreType.DMA,
            pltpu.SemaphoreType.DMA,
        ],
    )
    return k(feat_flat, membership, bounds2)


def kernel(atom_features, input_unused, membership):
    th = jnp.arange(W + 1, dtype=jnp.int32) * SPW
    b = jnp.sum(membership[None, :] < th[:, None], axis=1).astype(jnp.int32)
    bounds2 = jnp.zeros((W, 16), jnp.int32)
    bounds2 = bounds2.at[:, 0].set(b[:W]).at[:, 1].set(b[1:])
    out = _graph_gather(atom_features.reshape(-1), membership, bounds2)
    return out.reshape(SPAD, OD)[:S]


# speedup vs baseline: 1.0450x; 1.0450x over previous
import jax
import jax.numpy as jnp
from jax import lax
from jax.experimental import pallas as pl
from jax.experimental.pallas import tpu as pltpu
from jax.experimental.pallas import tpu_sc as plsc

N = 320000
D = 128
S = 10000
NC = 2
NS = 16
W = NC * NS
SPW = 320
SPAD = W * SPW
C = 80
B = 16
NBUF = 4
NSL = D // 16
OD = 2 * D


def _tree(vals, op):
    while len(vals) > 1:
        vals = [op(vals[t], vals[t + 1]) for t in range(0, len(vals), 2)]
    return vals[0]


def _sc_body(feat_hbm, mem_hbm, bnd_hbm, out_hbm, rows_v, mem_v, bnd_v, acc,
             st_s, st_m, cur_sm, sem_r, sem_m):
    wid = lax.axis_index("s") * NC + lax.axis_index("c")

    pltpu.sync_copy(bnd_hbm.at[wid], bnd_v)
    bvec = bnd_v[...]
    lo = bvec[0]
    hi = bvec[1]

    zero = jnp.zeros((16,), jnp.float32)
    CD = C * D
    CM = C + 16

    def issue(k, buf):
        pltpu.async_copy(feat_hbm.at[pl.ds(k * CD, CD)],
                         rows_v.at[pl.ds(buf * CD, CD)], sem_r)
        pltpu.async_copy(mem_hbm.at[pl.ds(k * C, C)],
                         mem_v.at[pl.ds(buf * CM, C)], sem_m)

    def wait(k, buf):
        pltpu.make_async_copy(feat_hbm.at[pl.ds(k * CD, CD)],
                              rows_v.at[pl.ds(buf * CD, CD)], sem_r).wait()
        pltpu.make_async_copy(mem_hbm.at[pl.ds(k * C, C)],
                              mem_v.at[pl.ds(buf * CM, C)], sem_m).wait()

    def flush_regs(cur, s_regs, m_regs):
        r = cur - wid * SPW
        for j in range(NSL):
            acc[pl.ds(r * OD + 16 * j, 16)] = jnp.maximum(s_regs[j], 0.0)
            acc[pl.ds(r * OD + D + 16 * j, 16)] = jnp.maximum(m_regs[j], 0.0)

    def flush_state(cur):
        flush_regs(cur,
                   [st_s[pl.ds(16 * j, 16)] for j in range(NSL)],
                   [st_m[pl.ds(16 * j, 16)] for j in range(NSL)])

    def make_row_step(rbase, mbase):
        def row_step(i, _):
            m = mem_v[pl.ds(mbase + i, 16)][0]
            cur = cur_sm[0]
            changed = m != cur

            @pl.when(jnp.logical_and(changed, cur >= 0))
            def _():
                flush_state(cur)

            for j in range(NSL):
                row = rows_v[pl.ds(rbase + i * D + 16 * j, 16)]
                s = st_s[pl.ds(16 * j, 16)]
                mx = st_m[pl.ds(16 * j, 16)]
                st_s[pl.ds(16 * j, 16)] = jnp.where(changed, row, s + row)
                st_m[pl.ds(16 * j, 16)] = jnp.where(
                    changed, row, jnp.maximum(mx, row))
            cur_sm[0] = m
            return 0
        return row_step

    def make_block_step(rbase, mbase):
        def block_step(ib, _):
            b0 = ib * B
            mvec = mem_v[pl.ds(mbase + b0, 16)]
            cur = cur_sm[0]
            last = mvec[15]

            @pl.when(last == cur)
            def _fast():
                for j in range(NSL):
                    vals = [rows_v[pl.ds(rbase + (b0 + i) * D + 16 * j, 16)]
                            for i in range(B)]
                    st_s[pl.ds(16 * j, 16)] = (
                        st_s[pl.ds(16 * j, 16)] + _tree(vals, lambda a, b: a + b))
                    st_m[pl.ds(16 * j, 16)] = jnp.maximum(
                        st_m[pl.ds(16 * j, 16)], _tree(vals, jnp.maximum))

            @pl.when(last != cur)
            def _slow():
                s_regs = [st_s[pl.ds(16 * j, 16)] for j in range(NSL)]
                m_regs = [st_m[pl.ds(16 * j, 16)] for j in range(NSL)]
                c = cur
                for i in range(B):
                    mi = mvec[i]
                    changed = mi != c
                    rows = [rows_v[pl.ds(rbase + (b0 + i) * D + 16 * j, 16)]
                            for j in range(NSL)]

                    @pl.when(jnp.logical_and(changed, c >= 0))
                    def _(c=c, s_regs=s_regs, m_regs=m_regs):
                        flush_regs(c, s_regs, m_regs)

                    s_regs = [jnp.where(changed, rows[j], s_regs[j] + rows[j])
                              for j in range(NSL)]
                    m_regs = [jnp.where(changed, rows[j],
                                        jnp.maximum(m_regs[j], rows[j]))
                              for j in range(NSL)]
                    c = mi
                for j in range(NSL):
                    st_s[pl.ds(16 * j, 16)] = s_regs[j]
                    st_m[pl.ds(16 * j, 16)] = m_regs[j]
                cur_sm[0] = last
            return 0
        return block_step

    def chunk_body(k, _):
        buf = k % NBUF
        start = k * C

        @pl.when(k + NBUF - 1 < c1)
        def _():
            issue(k + NBUF - 1, (k + NBUF - 1) % NBUF)

        wait(k, buf)

        i0 = jnp.maximum(lo, start) - start
        i1 = jnp.minimum(hi, start + C) - start
        rbase = buf * CD
        mbase = buf * CM
        row_step = make_row_step(rbase, mbase)
        block_step = make_block_step(rbase, mbase)
        lead_end = jnp.minimum(i1, ((i0 + B - 1) // B) * B)
        lax.fori_loop(i0, lead_end, row_step, 0)
        lax.fori_loop(lead_end // B, i1 // B, block_step, 0)
        lax.fori_loop(jnp.maximum(lead_end, (i1 // B) * B), i1, row_step, 0)
        return 0

    c0 = lo // C
    c1 = (hi + C - 1) // C

    for p in range(NBUF - 1):
        @pl.when(c0 + p < c1)
        def _(p=p):
            issue(c0 + p, (c0 + p) % NBUF)

    def zero_row(r, _):
        for j in range(NSL * 2):
            acc[pl.ds(r * OD + 16 * j, 16)] = zero
        return 0

    lax.fori_loop(0, SPW, zero_row, 0)
    cur_sm[0] = jnp.int32(-1)

    lax.fori_loop(c0, c1, chunk_body, 0)

    cur = cur_sm[0]

    @pl.when(cur >= 0)
    def _():
        flush_state(cur)

    LAST = S - (W - 1) * SPW

    @pl.when(wid < W - 1)
    def _():
        pltpu.sync_copy(acc.at[pl.ds(0, SPW * OD)],
                        out_hbm.at[pl.ds(wid * (SPW * OD), SPW * OD)])

    @pl.when(wid == W - 1)
    def _():
        pltpu.sync_copy(acc.at[pl.ds(0, LAST * OD)],
                        out_hbm.at[pl.ds(wid * (SPW * OD), LAST * OD)])


@jax.jit
def _graph_gather(feat_flat, membership, bounds2):
    mesh = plsc.VectorSubcoreMesh(
        core_axis_name="c", subcore_axis_name="s",
        num_cores=NC, num_subcores=NS)
    k = pl.kernel(
        _sc_body,
        out_type=jax.ShapeDtypeStruct((S * OD,), jnp.float32),
        mesh=mesh,
        scratch_types=[
            pltpu.VMEM((NBUF * C * D,), jnp.float32),
            pltpu.VMEM((NBUF * (C + 16),), jnp.int32),
            pltpu.VMEM((16,), jnp.int32),
            pltpu.VMEM((SPW * OD,), jnp.float32),
            pltpu.VMEM((D,), jnp.float32),
            pltpu.VMEM((D,), jnp.float32),
            pltpu.SMEM((8,), jnp.int32),
            pltpu.SemaphoreType.DMA,
            pltpu.SemaphoreType.DMA,
        ],
    )
    return k(feat_flat, membership, bounds2)


def kernel(atom_features, input_unused, membership):
    th = jnp.arange(W + 1, dtype=jnp.int32) * SPW
    b = jnp.sum(membership[None, :] < th[:, None], axis=1).astype(jnp.int32)
    bounds2 = jnp.zeros((W, 16), jnp.int32)
    bounds2 = bounds2.at[:, 0].set(b[:W]).at[:, 1].set(b[1:])
    out = _graph_gather(atom_features.reshape(-1), membership, bounds2)
    return out.reshape(S, OD)


# speedup vs baseline: 1.0591x; 1.0135x over previous
import jax
import jax.numpy as jnp
from jax import lax
from jax.experimental import pallas as pl
from jax.experimental.pallas import tpu as pltpu
from jax.experimental.pallas import tpu_sc as plsc

N = 320000
D = 128
S = 10000
NC = 2
NS = 16
W = NC * NS
SPW = 320
SPAD = W * SPW
C = 160
B = 16
NBUF = 2
NSL = D // 16
OD = 2 * D


def _tree(vals, op):
    while len(vals) > 1:
        vals = [op(vals[t], vals[t + 1]) for t in range(0, len(vals), 2)]
    return vals[0]


def _sc_body(feat_hbm, mem_hbm, bnd_hbm, out_hbm, rows_v, mem_v, bnd_v, acc,
             st_s, st_m, cur_sm, sem_r, sem_m):
    wid = lax.axis_index("s") * NC + lax.axis_index("c")

    pltpu.sync_copy(bnd_hbm.at[wid], bnd_v)
    bvec = bnd_v[...]
    lo = bvec[0]
    hi = bvec[1]

    zero = jnp.zeros((16,), jnp.float32)
    CD = C * D
    CM = C + 16

    def issue(k, buf):
        pltpu.async_copy(feat_hbm.at[pl.ds(k * CD, CD)],
                         rows_v.at[pl.ds(buf * CD, CD)], sem_r)
        pltpu.async_copy(mem_hbm.at[pl.ds(k * C, C)],
                         mem_v.at[pl.ds(buf * CM, C)], sem_m)

    def wait(k, buf):
        pltpu.make_async_copy(feat_hbm.at[pl.ds(k * CD, CD)],
                              rows_v.at[pl.ds(buf * CD, CD)], sem_r).wait()
        pltpu.make_async_copy(mem_hbm.at[pl.ds(k * C, C)],
                              mem_v.at[pl.ds(buf * CM, C)], sem_m).wait()

    def flush_regs(cur, s_regs, m_regs):
        r = cur - wid * SPW
        for j in range(NSL):
            acc[pl.ds(r * OD + 16 * j, 16)] = jnp.maximum(s_regs[j], 0.0)
            acc[pl.ds(r * OD + D + 16 * j, 16)] = jnp.maximum(m_regs[j], 0.0)

    def flush_state(cur):
        flush_regs(cur,
                   [st_s[pl.ds(16 * j, 16)] for j in range(NSL)],
                   [st_m[pl.ds(16 * j, 16)] for j in range(NSL)])

    def make_row_step(rbase, mbase):
        def row_step(i, _):
            m = mem_v[pl.ds(mbase + i, 16)][0]
            cur = cur_sm[0]
            changed = m != cur

            @pl.when(jnp.logical_and(changed, cur >= 0))
            def _():
                flush_state(cur)

            for j in range(NSL):
                row = rows_v[pl.ds(rbase + i * D + 16 * j, 16)]
                s = st_s[pl.ds(16 * j, 16)]
                mx = st_m[pl.ds(16 * j, 16)]
                st_s[pl.ds(16 * j, 16)] = jnp.where(changed, row, s + row)
                st_m[pl.ds(16 * j, 16)] = jnp.where(
                    changed, row, jnp.maximum(mx, row))
            cur_sm[0] = m
            return 0
        return row_step

    def make_block_step(rbase, mbase):
        def block_step(ib, _):
            b0 = ib * B
            mvec = mem_v[pl.ds(mbase + b0, 16)]
            cur = cur_sm[0]
            last = mvec[15]

            @pl.when(last == cur)
            def _fast():
                for j in range(NSL):
                    vals = [rows_v[pl.ds(rbase + (b0 + i) * D + 16 * j, 16)]
                            for i in range(B)]
                    st_s[pl.ds(16 * j, 16)] = (
                        st_s[pl.ds(16 * j, 16)] + _tree(vals, lambda a, b: a + b))
                    st_m[pl.ds(16 * j, 16)] = jnp.maximum(
                        st_m[pl.ds(16 * j, 16)], _tree(vals, jnp.maximum))

            @pl.when(last != cur)
            def _slow():
                s_regs = [st_s[pl.ds(16 * j, 16)] for j in range(NSL)]
                m_regs = [st_m[pl.ds(16 * j, 16)] for j in range(NSL)]
                c = cur
                for i in range(B):
                    mi = mvec[i]
                    changed = mi != c
                    rows = [rows_v[pl.ds(rbase + (b0 + i) * D + 16 * j, 16)]
                            for j in range(NSL)]

                    @pl.when(jnp.logical_and(changed, c >= 0))
                    def _(c=c, s_regs=s_regs, m_regs=m_regs):
                        flush_regs(c, s_regs, m_regs)

                    s_regs = [jnp.where(changed, rows[j], s_regs[j] + rows[j])
                              for j in range(NSL)]
                    m_regs = [jnp.where(changed, rows[j],
                                        jnp.maximum(m_regs[j], rows[j]))
                              for j in range(NSL)]
                    c = mi
                for j in range(NSL):
                    st_s[pl.ds(16 * j, 16)] = s_regs[j]
                    st_m[pl.ds(16 * j, 16)] = m_regs[j]
                cur_sm[0] = last
            return 0
        return block_step

    def chunk_body(k, _):
        buf = k % NBUF
        start = k * C

        @pl.when(k + NBUF - 1 < c1)
        def _():
            issue(k + NBUF - 1, (k + NBUF - 1) % NBUF)

        wait(k, buf)

        i0 = jnp.maximum(lo, start) - start
        i1 = jnp.minimum(hi, start + C) - start
        rbase = buf * CD
        mbase = buf * CM
        row_step = make_row_step(rbase, mbase)
        block_step = make_block_step(rbase, mbase)
        lead_end = jnp.minimum(i1, ((i0 + B - 1) // B) * B)
        lax.fori_loop(i0, lead_end, row_step, 0)
        lax.fori_loop(lead_end // B, i1 // B, block_step, 0)
        lax.fori_loop(jnp.maximum(lead_end, (i1 // B) * B), i1, row_step, 0)
        return 0

    c0 = lo // C
    c1 = (hi + C - 1) // C

    for p in range(NBUF - 1):
        @pl.when(c0 + p < c1)
        def _(p=p):
            issue(c0 + p, (c0 + p) % NBUF)

    def zero_row(r, _):
        for j in range(NSL * 2):
            acc[pl.ds(r * OD + 16 * j, 16)] = zero
        return 0

    lax.fori_loop(0, SPW, zero_row, 0)
    cur_sm[0] = jnp.int32(-1)

    lax.fori_loop(c0, c1, chunk_body, 0)

    cur = cur_sm[0]

    @pl.when(cur >= 0)
    def _():
        flush_state(cur)

    LAST = S - (W - 1) * SPW

    @pl.when(wid < W - 1)
    def _():
        pltpu.sync_copy(acc.at[pl.ds(0, SPW * OD)],
                        out_hbm.at[pl.ds(wid * (SPW * OD), SPW * OD)])

    @pl.when(wid == W - 1)
    def _():
        pltpu.sync_copy(acc.at[pl.ds(0, LAST * OD)],
                        out_hbm.at[pl.ds(wid * (SPW * OD), LAST * OD)])


@jax.jit
def _graph_gather(feat_flat, membership, bounds2):
    mesh = plsc.VectorSubcoreMesh(
        core_axis_name="c", subcore_axis_name="s",
        num_cores=NC, num_subcores=NS)
    k = pl.kernel(
        _sc_body,
        out_type=jax.ShapeDtypeStruct((S * OD,), jnp.float32),
        mesh=mesh,
        scratch_types=[
            pltpu.VMEM((NBUF * C * D,), jnp.float32),
            pltpu.VMEM((NBUF * (C + 16),), jnp.int32),
            pltpu.VMEM((16,), jnp.int32),
            pltpu.VMEM((SPW * OD,), jnp.float32),
            pltpu.VMEM((D,), jnp.float32),
            pltpu.VMEM((D,), jnp.float32),
            pltpu.SMEM((8,), jnp.int32),
            pltpu.SemaphoreType.DMA,
            pltpu.SemaphoreType.DMA,
        ],
    )
    return k(feat_flat, membership, bounds2)


def kernel(atom_features, input_unused, membership):
    th = jnp.arange(W + 1, dtype=jnp.int32) * SPW
    b = jnp.sum(membership[None, :] < th[:, None], axis=1).astype(jnp.int32)
    bounds2 = jnp.zeros((W, 16), jnp.int32)
    bounds2 = bounds2.at[:, 0].set(b[:W]).at[:, 1].set(b[1:])
    out = _graph_gather(atom_features.reshape(-1), membership, bounds2)
    return out.reshape(S, OD)
